# baseline (speedup 1.0000x reference)
import functools

import jax
import jax.numpy as jnp
from jax.experimental import pallas as pl
from jax.experimental.pallas import tpu as pltpu

_MB = 128
_TN = 2048
_NEG = float("-inf")
_INF = float("inf")


def _lane_fold(v):
    parts = [v[:, j * 128:(j + 1) * 128] for j in range(v.shape[1] // 128)]
    acc = parts[0]
    for p in parts[1:]:
        acc = jnp.minimum(acc, p)
    return acc


def _lane_fold_add(v):
    parts = [v[:, j * 128:(j + 1) * 128] for j in range(v.shape[1] // 128)]
    acc = parts[0]
    for p in parts[1:]:
        acc = acc + p
    return acc


def _knn_interp_kernel(k_ref, pos_y_ref, by_ref, pos_xt_ref, bx_ref,
                       xh_ref, xl_ref, out_ref, d2_ref, *, n, d):
    nt = n // _TN
    kdyn = k_ref[0]
    yx = pos_y_ref[:, 0:1]
    yy = pos_y_ref[:, 1:2]
    yz = pos_y_ref[:, 2:3]
    by = by_ref[:, 0:1]
    ynorm = yx * yx + yy * yy + yz * yz
    yb = pos_y_ref[...].astype(jnp.bfloat16)

    def _dist(i, m):
        off = i * _TN
        xs = pos_xt_ref[:, pl.ds(off, _TN)]
        x0 = xs[0:1, :]
        x1 = xs[1:2, :]
        x2 = xs[2:3, :]
        xnorm = x0 * x0 + x1 * x1 + x2 * x2
        mm = jnp.dot(yb, xs.astype(jnp.bfloat16),
                     preferred_element_type=jnp.float32)
        d2 = (ynorm + xnorm) - 2.0 * mm
        valid = by == bx_ref[:, pl.ds(off, _TN)]
        d2m = jnp.where(valid, d2, _INF)
        d2_ref[:, pl.ds(off, _TN)] = d2m
        return jnp.minimum(m, _lane_fold(d2m))

    acc = jax.lax.fori_loop(0, nt, _dist,
                            jnp.full((_MB, 128), _INF, dtype=jnp.float32),
                            unroll=2)
    t = jnp.min(acc, axis=1, keepdims=True)

    kf = kdyn.astype(jnp.float32)
    for _ in range(1, 8):
        def _round(i, carry):
            m, cacc = carry
            v = d2_ref[:, pl.ds(i * _TN, _TN)]
            le = v <= t
            vm = jnp.where(le, _INF, v)
            ones = jnp.where(le, 1.0, 0.0)
            return (jnp.minimum(m, _lane_fold(vm)),
                    cacc + _lane_fold_add(ones))

        acc, cacc = jax.lax.fori_loop(
            0, nt, _round,
            (jnp.full((_MB, 128), _INF, dtype=jnp.float32),
             jnp.zeros((_MB, 128), dtype=jnp.float32)),
            unroll=4)
        nxt = jnp.min(acc, axis=1, keepdims=True)
        cnt = jnp.sum(cacc, axis=1, keepdims=True)
        t = jnp.where(cnt < kf, nxt, t)

    def _tiescan(i, carry):
        i1, i2, cl = carry
        off = i * _TN
        v = d2_ref[:, pl.ds(off, _TN)]
        lane = jax.lax.broadcasted_iota(jnp.int32, (_MB, _TN), 1)
        idx = (lane + off).astype(jnp.float32)
        idxv = jnp.where(v == t, idx, _INF)
        lt = jnp.where(v < t, 1.0, 0.0)
        f1 = _lane_fold(idxv)
        return (jnp.minimum(i1, f1),
                jnp.minimum(i2, jnp.maximum(i1, f1)),
                cl + _lane_fold_add(lt))

    i1a, i2a, cla = jax.lax.fori_loop(
        0, nt, _tiescan,
        (jnp.full((_MB, 128), _INF, dtype=jnp.float32),
         jnp.full((_MB, 128), _INF, dtype=jnp.float32),
         jnp.zeros((_MB, 128), dtype=jnp.float32)),
        unroll=4)
    gi1 = jnp.min(i1a, axis=1, keepdims=True)
    gi2 = jnp.minimum(jnp.min(jnp.where(i1a == gi1, _INF, i1a),
                              axis=1, keepdims=True),
                      jnp.min(i2a, axis=1, keepdims=True))
    cnt_lt = jnp.sum(cla, axis=1, keepdims=True)
    need = kf - cnt_lt
    t_idx = jnp.where(need <= 1.0, gi1, jnp.where(need <= 2.0, gi2, _INF))

    def _combine(i, carry):
        num, den = carry
        off = i * _TN
        v = d2_ref[:, pl.ds(off, _TN)]
        xs = pos_xt_ref[:, pl.ds(off, _TN)]
        d2e = ((yx - xs[0:1, :]) ** 2
               + (yy - xs[1:2, :]) ** 2
               + (yz - xs[2:3, :]) ** 2)
        lane = jax.lax.broadcasted_iota(jnp.int32, (_MB, _TN), 1)
        idx = (lane + off).astype(jnp.float32)
        sel = (v < t) | ((v == t) & (idx <= t_idx))
        w = jnp.where(sel, 1.0 / jnp.maximum(d2e, 1e-16), 0.0)
        den = den + jnp.sum(w, axis=1, keepdims=True)
        w_hi = w.astype(jnp.bfloat16)
        w_lo = (w - w_hi.astype(jnp.float32)).astype(jnp.bfloat16)
        xh = xh_ref[pl.ds(off, _TN), :]
        xl = xl_ref[pl.ds(off, _TN), :]
        num = (num
               + jnp.dot(w_hi, xh, preferred_element_type=jnp.float32)
               + jnp.dot(w_hi, xl, preferred_element_type=jnp.float32)
               + jnp.dot(w_lo, xh, preferred_element_type=jnp.float32))
        return num, den

    num0 = jnp.zeros((_MB, d), dtype=jnp.float32)
    den0 = jnp.zeros((_MB, 1), dtype=jnp.float32)
    num, den = jax.lax.fori_loop(0, nt, _combine, (num0, den0), unroll=2)
    out_ref[...] = num / den


def kernel(x, pos_x, pos_y, batch_x, batch_y, k):
    n, d = x.shape
    m = pos_y.shape[0]
    karr = jnp.asarray(k, dtype=jnp.int32).reshape(1)
    pos_xt = pos_x.T
    bx = batch_x.astype(jnp.int32).reshape(1, n)
    by = batch_y.astype(jnp.int32).reshape(m, 1)
    x_hi = x.astype(jnp.bfloat16)
    x_lo = (x - x_hi.astype(jnp.float32)).astype(jnp.bfloat16)

    grid = (m // _MB,)
    out = pl.pallas_call(
        functools.partial(_knn_interp_kernel, n=n, d=d),
        grid=grid,
        in_specs=[
            pl.BlockSpec(memory_space=pltpu.SMEM),
            pl.BlockSpec((_MB, 3), lambda i: (i, 0)),
            pl.BlockSpec((_MB, 1), lambda i: (i, 0)),
            pl.BlockSpec((3, n), lambda i: (0, 0)),
            pl.BlockSpec((1, n), lambda i: (0, 0)),
            pl.BlockSpec((n, d), lambda i: (0, 0)),
            pl.BlockSpec((n, d), lambda i: (0, 0)),
        ],
        out_specs=pl.BlockSpec((_MB, d), lambda i: (i, 0)),
        out_shape=jax.ShapeDtypeStruct((m, d), jnp.float32),
        scratch_shapes=[pltpu.VMEM((_MB, n), jnp.float32)],
    )(karr, pos_y, by, pos_xt, bx, x_hi, x_lo)
    return out


# speedup vs baseline: 1.1600x; 1.1600x over previous
import functools

import jax
import jax.numpy as jnp
from jax.experimental import pallas as pl
from jax.experimental.pallas import tpu as pltpu

_MB = 128
_TN = 2048
_NEG = float("-inf")
_INF = float("inf")


def _lane_fold(v):
    parts = [v[:, j * 128:(j + 1) * 128] for j in range(v.shape[1] // 128)]
    acc = parts[0]
    for p in parts[1:]:
        acc = jnp.minimum(acc, p)
    return acc


def _lane_fold_add(v):
    parts = [v[:, j * 128:(j + 1) * 128] for j in range(v.shape[1] // 128)]
    acc = parts[0]
    for p in parts[1:]:
        acc = acc + p
    return acc


def _knn_interp_kernel(k_ref, pos_y_ref, by_ref, pos_xt_ref, bx_ref,
                       xh_ref, xl_ref, out_ref, d2_ref, t_ref, *, n, d):
    nt = n // _TN
    kdyn = k_ref[0]
    yx = pos_y_ref[:, 0:1]
    yy = pos_y_ref[:, 1:2]
    yz = pos_y_ref[:, 2:3]
    by = by_ref[:, 0:1]
    ynorm = yx * yx + yy * yy + yz * yz
    yb = pos_y_ref[...].astype(jnp.bfloat16)

    def _dist(i, m):
        off = i * _TN
        xs = pos_xt_ref[:, pl.ds(off, _TN)]
        x0 = xs[0:1, :]
        x1 = xs[1:2, :]
        x2 = xs[2:3, :]
        xnorm = x0 * x0 + x1 * x1 + x2 * x2
        mm = jnp.dot(yb, xs.astype(jnp.bfloat16),
                     preferred_element_type=jnp.float32)
        d2 = (ynorm + xnorm) - 2.0 * mm
        valid = by == bx_ref[:, pl.ds(off, _TN)]
        d2m = jnp.where(valid, d2, _INF)
        d2_ref[:, pl.ds(off, _TN)] = d2m
        return jnp.minimum(m, _lane_fold(d2m))

    acc = jax.lax.fori_loop(0, nt, _dist,
                            jnp.full((_MB, 128), _INF, dtype=jnp.float32),
                            unroll=2)
    t = jnp.min(acc, axis=1, keepdims=True)

    kf = kdyn.astype(jnp.float32)
    t0 = t

    t_sel = jnp.where(0 < kdyn, t, _NEG)
    for r in range(1, 8):
        def _round(i, m):
            v = d2_ref[:, pl.ds(i * _TN, _TN)]
            vm = jnp.where(v <= t, _INF, v)
            return jnp.minimum(m, _lane_fold(vm))

        acc = jax.lax.fori_loop(0, nt, _round,
                                jnp.full((_MB, 128), _INF, dtype=jnp.float32),
                                unroll=4)
        t = jnp.min(acc, axis=1, keepdims=True)
        t_sel = jnp.where(r < kdyn, t, t_sel)

    def _vcount(i, c):
        v = d2_ref[:, pl.ds(i * _TN, _TN)]
        return c + _lane_fold_add(jnp.where(v <= t_sel, 1.0, 0.0))

    cacc = jax.lax.fori_loop(0, nt, _vcount,
                             jnp.zeros((_MB, 128), dtype=jnp.float32),
                             unroll=4)
    cnt8 = jnp.sum(cacc, axis=1, keepdims=True)
    nbad = jnp.sum(jnp.where(cnt8 > kf, 1.0, 0.0))
    t_ref[:, 0:1] = t_sel

    @pl.when(nbad > 0.0)
    def _slow():
        ts = t0
        for _ in range(1, 8):
            def _round(i, carry):
                m, cacc = carry
                v = d2_ref[:, pl.ds(i * _TN, _TN)]
                le = v <= ts
                vm = jnp.where(le, _INF, v)
                ones = jnp.where(le, 1.0, 0.0)
                return (jnp.minimum(m, _lane_fold(vm)),
                        cacc + _lane_fold_add(ones))

            acc, ca = jax.lax.fori_loop(
                0, nt, _round,
                (jnp.full((_MB, 128), _INF, dtype=jnp.float32),
                 jnp.zeros((_MB, 128), dtype=jnp.float32)),
                unroll=4)
            nxt = jnp.min(acc, axis=1, keepdims=True)
            cnt = jnp.sum(ca, axis=1, keepdims=True)
            ts = jnp.where(cnt < kf, nxt, ts)
        t_ref[:, 0:1] = ts

    t = t_ref[:, 0:1]

    def _tiescan(i, carry):
        i1, i2, cl = carry
        off = i * _TN
        v = d2_ref[:, pl.ds(off, _TN)]
        lane = jax.lax.broadcasted_iota(jnp.int32, (_MB, _TN), 1)
        idx = (lane + off).astype(jnp.float32)
        idxv = jnp.where(v == t, idx, _INF)
        lt = jnp.where(v < t, 1.0, 0.0)
        f1 = _lane_fold(idxv)
        return (jnp.minimum(i1, f1),
                jnp.minimum(i2, jnp.maximum(i1, f1)),
                cl + _lane_fold_add(lt))

    i1a, i2a, cla = jax.lax.fori_loop(
        0, nt, _tiescan,
        (jnp.full((_MB, 128), _INF, dtype=jnp.float32),
         jnp.full((_MB, 128), _INF, dtype=jnp.float32),
         jnp.zeros((_MB, 128), dtype=jnp.float32)),
        unroll=4)
    gi1 = jnp.min(i1a, axis=1, keepdims=True)
    gi2 = jnp.minimum(jnp.min(jnp.where(i1a == gi1, _INF, i1a),
                              axis=1, keepdims=True),
                      jnp.min(i2a, axis=1, keepdims=True))
    cnt_lt = jnp.sum(cla, axis=1, keepdims=True)
    need = kf - cnt_lt
    t_idx = jnp.where(need <= 1.0, gi1, jnp.where(need <= 2.0, gi2, _INF))

    def _combine(i, carry):
        num, den = carry
        off = i * _TN
        v = d2_ref[:, pl.ds(off, _TN)]
        xs = pos_xt_ref[:, pl.ds(off, _TN)]
        d2e = ((yx - xs[0:1, :]) ** 2
               + (yy - xs[1:2, :]) ** 2
               + (yz - xs[2:3, :]) ** 2)
        lane = jax.lax.broadcasted_iota(jnp.int32, (_MB, _TN), 1)
        idx = (lane + off).astype(jnp.float32)
        sel = (v < t) | ((v == t) & (idx <= t_idx))
        w = jnp.where(sel, 1.0 / jnp.maximum(d2e, 1e-16), 0.0)
        den = den + jnp.sum(w, axis=1, keepdims=True)
        w_hi = w.astype(jnp.bfloat16)
        w_lo = (w - w_hi.astype(jnp.float32)).astype(jnp.bfloat16)
        xh = xh_ref[pl.ds(off, _TN), :]
        xl = xl_ref[pl.ds(off, _TN), :]
        num = (num
               + jnp.dot(w_hi, xh, preferred_element_type=jnp.float32)
               + jnp.dot(w_hi, xl, preferred_element_type=jnp.float32)
               + jnp.dot(w_lo, xh, preferred_element_type=jnp.float32))
        return num, den

    num0 = jnp.zeros((_MB, d), dtype=jnp.float32)
    den0 = jnp.zeros((_MB, 1), dtype=jnp.float32)
    num, den = jax.lax.fori_loop(0, nt, _combine, (num0, den0), unroll=2)
    out_ref[...] = num / den


def kernel(x, pos_x, pos_y, batch_x, batch_y, k):
    n, d = x.shape
    m = pos_y.shape[0]
    karr = jnp.asarray(k, dtype=jnp.int32).reshape(1)
    pos_xt = pos_x.T
    bx = batch_x.astype(jnp.int32).reshape(1, n)
    by = batch_y.astype(jnp.int32).reshape(m, 1)
    x_hi = x.astype(jnp.bfloat16)
    x_lo = (x - x_hi.astype(jnp.float32)).astype(jnp.bfloat16)

    grid = (m // _MB,)
    out = pl.pallas_call(
        functools.partial(_knn_interp_kernel, n=n, d=d),
        grid=grid,
        in_specs=[
            pl.BlockSpec(memory_space=pltpu.SMEM),
            pl.BlockSpec((_MB, 3), lambda i: (i, 0)),
            pl.BlockSpec((_MB, 1), lambda i: (i, 0)),
            pl.BlockSpec((3, n), lambda i: (0, 0)),
            pl.BlockSpec((1, n), lambda i: (0, 0)),
            pl.BlockSpec((n, d), lambda i: (0, 0)),
            pl.BlockSpec((n, d), lambda i: (0, 0)),
        ],
        out_specs=pl.BlockSpec((_MB, d), lambda i: (i, 0)),
        out_shape=jax.ShapeDtypeStruct((m, d), jnp.float32),
        scratch_shapes=[pltpu.VMEM((_MB, n), jnp.float32),
                        pltpu.VMEM((_MB, 128), jnp.float32)],
    )(karr, pos_y, by, pos_xt, bx, x_hi, x_lo)
    return out


# speedup vs baseline: 1.1620x; 1.0018x over previous
import functools

import jax
import jax.numpy as jnp
from jax.experimental import pallas as pl
from jax.experimental.pallas import tpu as pltpu

_MB = 128
_TN = 2048
_NEG = float("-inf")
_INF = float("inf")


def _lane_fold(v):
    parts = [v[:, j * 128:(j + 1) * 128] for j in range(v.shape[1] // 128)]
    acc = parts[0]
    for p in parts[1:]:
        acc = jnp.minimum(acc, p)
    return acc


def _lane_fold_add(v):
    parts = [v[:, j * 128:(j + 1) * 128] for j in range(v.shape[1] // 128)]
    acc = parts[0]
    for p in parts[1:]:
        acc = acc + p
    return acc


def _knn_interp_kernel(k_ref, pos_y_ref, by_ref, pos_xt_ref, bx_ref,
                       xh_ref, xl_ref, out_ref, d2_ref, t_ref, *, n, d):
    nt = n // _TN
    kdyn = k_ref[0]
    yx = pos_y_ref[:, 0:1]
    yy = pos_y_ref[:, 1:2]
    yz = pos_y_ref[:, 2:3]
    by = by_ref[:, 0:1]
    ynorm = yx * yx + yy * yy + yz * yz
    yb = pos_y_ref[...].astype(jnp.bfloat16)

    def _dist(i, m):
        off = i * _TN
        xs = pos_xt_ref[:, pl.ds(off, _TN)]
        x0 = xs[0:1, :]
        x1 = xs[1:2, :]
        x2 = xs[2:3, :]
        xnorm = x0 * x0 + x1 * x1 + x2 * x2
        mm = jnp.dot(yb, xs.astype(jnp.bfloat16),
                     preferred_element_type=jnp.float32)
        d2 = (ynorm + xnorm) - 2.0 * mm
        valid = by == bx_ref[:, pl.ds(off, _TN)]
        d2m = jnp.where(valid, d2, _INF)
        d2_ref[:, pl.ds(off, _TN)] = d2m
        return jnp.minimum(m, _lane_fold(d2m))

    acc = jax.lax.fori_loop(0, nt, _dist,
                            jnp.full((_MB, 128), _INF, dtype=jnp.float32),
                            unroll=2)
    t = jnp.min(acc, axis=1, keepdims=True)

    kf = kdyn.astype(jnp.float32)
    t0 = t

    t_sel = jnp.where(0 < kdyn, t, _NEG)
    for r in range(1, 8):
        def _round(i, m):
            v = d2_ref[:, pl.ds(i * _TN, _TN)]
            vm = jnp.where(v <= t, _INF, v)
            return jnp.minimum(m, _lane_fold(vm))

        acc = jax.lax.fori_loop(0, nt, _round,
                                jnp.full((_MB, 128), _INF, dtype=jnp.float32),
                                unroll=4)
        t = jnp.min(acc, axis=1, keepdims=True)
        t_sel = jnp.where(r < kdyn, t, t_sel)

    def _tie_scan(tt):
        def _body(i, carry):
            i1, i2, cl, ce = carry
            off = i * _TN
            v = d2_ref[:, pl.ds(off, _TN)]
            lane = jax.lax.broadcasted_iota(jnp.int32, (_MB, _TN), 1)
            idx = (lane + off).astype(jnp.float32)
            eq = v == tt
            idxv = jnp.where(eq, idx, _INF)
            f1 = _lane_fold(idxv)
            return (jnp.minimum(i1, f1),
                    jnp.minimum(i2, jnp.maximum(i1, f1)),
                    cl + _lane_fold_add(jnp.where(v < tt, 1.0, 0.0)),
                    ce + _lane_fold_add(jnp.where(eq, 1.0, 0.0)))

        i1a, i2a, cla, cea = jax.lax.fori_loop(
            0, nt, _body,
            (jnp.full((_MB, 128), _INF, dtype=jnp.float32),
             jnp.full((_MB, 128), _INF, dtype=jnp.float32),
             jnp.zeros((_MB, 128), dtype=jnp.float32),
             jnp.zeros((_MB, 128), dtype=jnp.float32)),
            unroll=4)
        g1 = jnp.min(i1a, axis=1, keepdims=True)
        g2 = jnp.minimum(jnp.min(jnp.where(i1a == g1, _INF, i1a),
                                 axis=1, keepdims=True),
                         jnp.min(i2a, axis=1, keepdims=True))
        clt = jnp.sum(cla, axis=1, keepdims=True)
        ceq = jnp.sum(cea, axis=1, keepdims=True)
        return g1, g2, clt, ceq

    g1, g2, clt, ceq = _tie_scan(t_sel)
    nbad = jnp.sum(jnp.where(clt + ceq > kf, 1.0, 0.0))
    t_ref[:, 0:1] = t_sel
    t_ref[:, 1:2] = g1
    t_ref[:, 2:3] = g2
    t_ref[:, 3:4] = clt

    @pl.when(nbad > 0.0)
    def _slow():
        ts = t0
        for _ in range(1, 8):
            def _round(i, carry):
                m, cacc = carry
                v = d2_ref[:, pl.ds(i * _TN, _TN)]
                le = v <= ts
                vm = jnp.where(le, _INF, v)
                ones = jnp.where(le, 1.0, 0.0)
                return (jnp.minimum(m, _lane_fold(vm)),
                        cacc + _lane_fold_add(ones))

            acc, ca = jax.lax.fori_loop(
                0, nt, _round,
                (jnp.full((_MB, 128), _INF, dtype=jnp.float32),
                 jnp.zeros((_MB, 128), dtype=jnp.float32)),
                unroll=4)
            nxt = jnp.min(acc, axis=1, keepdims=True)
            cnt = jnp.sum(ca, axis=1, keepdims=True)
            ts = jnp.where(cnt < kf, nxt, ts)
        sg1, sg2, sclt, _ = _tie_scan(ts)
        t_ref[:, 0:1] = ts
        t_ref[:, 1:2] = sg1
        t_ref[:, 2:3] = sg2
        t_ref[:, 3:4] = sclt

    t = t_ref[:, 0:1]
    gi1 = t_ref[:, 1:2]
    gi2 = t_ref[:, 2:3]
    cnt_lt = t_ref[:, 3:4]
    need = kf - cnt_lt
    t_idx = jnp.where(need <= 1.0, gi1, jnp.where(need <= 2.0, gi2, _INF))

    def _combine(i, carry):
        num, den = carry
        off = i * _TN
        v = d2_ref[:, pl.ds(off, _TN)]
        xs = pos_xt_ref[:, pl.ds(off, _TN)]
        d2e = ((yx - xs[0:1, :]) ** 2
               + (yy - xs[1:2, :]) ** 2
               + (yz - xs[2:3, :]) ** 2)
        lane = jax.lax.broadcasted_iota(jnp.int32, (_MB, _TN), 1)
        idx = (lane + off).astype(jnp.float32)
        sel = (v < t) | ((v == t) & (idx <= t_idx))
        w = jnp.where(sel, 1.0 / jnp.maximum(d2e, 1e-16), 0.0)
        den = den + jnp.sum(w, axis=1, keepdims=True)
        w_hi = w.astype(jnp.bfloat16)
        w_lo = (w - w_hi.astype(jnp.float32)).astype(jnp.bfloat16)
        xh = xh_ref[pl.ds(off, _TN), :]
        xl = xl_ref[pl.ds(off, _TN), :]
        num = (num
               + jnp.dot(w_hi, xh, preferred_element_type=jnp.float32)
               + jnp.dot(w_hi, xl, preferred_element_type=jnp.float32)
               + jnp.dot(w_lo, xh, preferred_element_type=jnp.float32))
        return num, den

    num0 = jnp.zeros((_MB, d), dtype=jnp.float32)
    den0 = jnp.zeros((_MB, 1), dtype=jnp.float32)
    num, den = jax.lax.fori_loop(0, nt, _combine, (num0, den0), unroll=2)
    out_ref[...] = num / den


def kernel(x, pos_x, pos_y, batch_x, batch_y, k):
    n, d = x.shape
    m = pos_y.shape[0]
    karr = jnp.asarray(k, dtype=jnp.int32).reshape(1)
    pos_xt = pos_x.T
    bx = batch_x.astype(jnp.int32).reshape(1, n)
    by = batch_y.astype(jnp.int32).reshape(m, 1)
    x_hi = x.astype(jnp.bfloat16)
    x_lo = (x - x_hi.astype(jnp.float32)).astype(jnp.bfloat16)

    grid = (m // _MB,)
    out = pl.pallas_call(
        functools.partial(_knn_interp_kernel, n=n, d=d),
        grid=grid,
        in_specs=[
            pl.BlockSpec(memory_space=pltpu.SMEM),
            pl.BlockSpec((_MB, 3), lambda i: (i, 0)),
            pl.BlockSpec((_MB, 1), lambda i: (i, 0)),
            pl.BlockSpec((3, n), lambda i: (0, 0)),
            pl.BlockSpec((1, n), lambda i: (0, 0)),
            pl.BlockSpec((n, d), lambda i: (0, 0)),
            pl.BlockSpec((n, d), lambda i: (0, 0)),
        ],
        out_specs=pl.BlockSpec((_MB, d), lambda i: (i, 0)),
        out_shape=jax.ShapeDtypeStruct((m, d), jnp.float32),
        scratch_shapes=[pltpu.VMEM((_MB, n), jnp.float32),
                        pltpu.VMEM((_MB, 128), jnp.float32)],
    )(karr, pos_y, by, pos_xt, bx, x_hi, x_lo)
    return out
